# Initial kernel scaffold; baseline (speedup 1.0000x reference)
import jax
import jax.numpy as jnp
from jax.experimental import pallas as pl


def kernel(x, edge_index, W1, b1, W2, b2, p):
    raise NotImplementedError("write your pallas kernel here")


# speedup vs baseline: 22.0310x; 22.0310x over previous
import functools

import jax
import jax.numpy as jnp
from jax import lax
from jax.experimental import pallas as pl
from jax.experimental.pallas import tpu as pltpu
from jax.experimental.pallas import tpu_sc as plsc

N = 10000
NFEAT = 128
NHID = 128
NCLASS = 64
E = 320000

NC = 2
NS = 16
NW = NC * NS
K = 128

N1 = 10112
RPT = N1 // NS
ZR = 64
_CHUNKS = [(t * ZR, ZR) for t in range(RPT // ZR)] + [(RPT - RPT % ZR, RPT % ZR)]

EP = E + N
NCH = -(-EP // (NW * K))
EPAD = NW * NCH * K

_mesh = plsc.VectorSubcoreMesh(core_axis_name="c", subcore_axis_name="s")


@functools.partial(
    pl.kernel,
    mesh=_mesh,
    out_type=jax.ShapeDtypeStruct((NC * N1,), jnp.float32),
    scratch_types=[
        pltpu.VMEM((NCH, K), jnp.int32),
        pltpu.VMEM((K,), jnp.float32),
        pltpu.VMEM((RPT,), jnp.float32),
        pltpu.VMEM_SHARED((N1,), jnp.float32),
    ],
)
def _deg_kernel(dsts, out, dst_v, ones_v, zld, deg_sh):
    c = lax.axis_index("c")
    s = lax.axis_index("s")
    wid = c * NS + s

    for j in range(K // 16):
        ones_v[pl.ds(j * 16, 16)] = jnp.full((16,), 1.0, jnp.float32)

    def zfill(i, carry):
        zld[pl.ds(i * 16, 16)] = jnp.zeros((16,), jnp.float32)
        return carry

    lax.fori_loop(0, RPT // 16, zfill, 0)
    zld[pl.ds(RPT - 16, 16)] = jnp.zeros((16,), jnp.float32)

    pltpu.sync_copy(zld, deg_sh.at[pl.ds(s * RPT, RPT)])

    pltpu.sync_copy(dsts.at[wid], dst_v)
    plsc.subcore_barrier()

    def body(j, carry):
        pltpu.sync_copy(ones_v, deg_sh.at[dst_v.at[j]], add=True)
        return carry

    lax.fori_loop(0, NCH, body, 0)
    plsc.subcore_barrier()
    pltpu.sync_copy(deg_sh.at[pl.ds(s * RPT, RPT)], zld)
    pltpu.sync_copy(zld, out.at[pl.ds(c * N1 + s * RPT, RPT)])


def _make_scatter_kernel(D):
    @functools.partial(
        pl.kernel,
        mesh=_mesh,
        out_type=jax.ShapeDtypeStruct((NC, N1, D), jnp.float32),
        scratch_types=[
            pltpu.VMEM((NCH, K), jnp.int32),
            pltpu.VMEM((NCH, K), jnp.int32),
            pltpu.VMEM((K, D), jnp.float32),
            pltpu.VMEM((ZR, D), jnp.float32),
            pltpu.VMEM_SHARED((N1, D), jnp.float32),
            pltpu.SemaphoreType.DMA,
        ],
    )
    def k(table, srcs, dsts, out, src_v, dst_v, rows_v, zv, acc, sem):
        c = lax.axis_index("c")
        s = lax.axis_index("s")
        wid = c * NS + s

        def zfill(i, carry):
            for j in range(D // 16):
                zv[i, pl.ds(j * 16, 16)] = jnp.zeros((16,), jnp.float32)
            return carry

        lax.fori_loop(0, ZR, zfill, 0)
        for off, n in _CHUNKS:
            pltpu.sync_copy(zv.at[pl.ds(0, n)],
                            acc.at[pl.ds(s * RPT + off, n)])

        pltpu.sync_copy(srcs.at[wid], src_v)
        pltpu.sync_copy(dsts.at[wid], dst_v)
        plsc.subcore_barrier()

        def body(j, carry):
            pltpu.async_copy(table.at[src_v.at[j]], rows_v, sem).wait()
            pltpu.sync_copy(rows_v, acc.at[dst_v.at[j]], add=True)
            return carry

        lax.fori_loop(0, NCH, body, 0)
        plsc.subcore_barrier()
        for off, n in _CHUNKS:
            pltpu.sync_copy(acc.at[pl.ds(s * RPT + off, n)], zv.at[pl.ds(0, n)])
            pltpu.sync_copy(zv.at[pl.ds(0, n)], out.at[c, pl.ds(s * RPT + off, n)])

    return k


_scatter128 = _make_scatter_kernel(NHID)


def _tc1_body(degp_ref, x_ref, w1_ref, h1s_ref, dis_ref):
    deg = degp_ref[0] + degp_ref[1]
    dis = jnp.where(deg > 0.0, lax.rsqrt(deg), 0.0)
    h = jnp.dot(x_ref[...], w1_ref[...], preferred_element_type=jnp.float32)
    h1s_ref[...] = h * dis
    dis_ref[...] = dis


def _tc2_body(accp_ref, dis_ref, b1_ref, p_ref, w2_ref, h2s_ref):
    a = accp_ref[0] + accp_ref[1]
    dis = dis_ref[...]
    z = jnp.maximum(a * dis + b1_ref[...], 0.0) * jnp.clip(p_ref[...], 0.0, 1.0)
    row = lax.broadcasted_iota(jnp.int32, (N1, 1), 0)
    z = jnp.where(row < N, z, 0.0)
    h2s_ref[...] = jnp.dot(z, w2_ref[...],
                           preferred_element_type=jnp.float32) * dis


def _tc3_body(accp_ref, dis_ref, b2_ref, out_ref):
    a = accp_ref[0] + accp_ref[1]
    full = a * dis_ref[...] + b2_ref[...]
    out_ref[...] = full[:N, :]


_tc1 = pl.pallas_call(
    _tc1_body,
    out_shape=(jax.ShapeDtypeStruct((N1, NHID), jnp.float32),
               jax.ShapeDtypeStruct((N1, 1), jnp.float32)),
)

_tc2 = pl.pallas_call(
    _tc2_body,
    out_shape=jax.ShapeDtypeStruct((N1, NHID), jnp.float32),
)

_tc3 = pl.pallas_call(
    _tc3_body,
    out_shape=jax.ShapeDtypeStruct((N, NHID), jnp.float32),
)


def kernel(x, edge_index, W1, b1, W2, b2, p):
    ei = edge_index.astype(jnp.int32)
    src = ei[0]
    dst = ei[1]
    loop = jnp.arange(N, dtype=jnp.int32)
    npad = EPAD - EP
    pad_i = jnp.arange(npad, dtype=jnp.int32)
    pad_src = (pad_i * 997) % N
    pad_dst = N + pad_i % (N1 - N)
    srcs = jnp.concatenate([src, loop, pad_src]).reshape(NW, NCH, K)
    dsts = jnp.concatenate([dst, loop, pad_dst]).reshape(NW, NCH, K)
    x_pad = jnp.pad(x, ((0, N1 - N), (0, 0)))

    W2p = jnp.pad(W2, ((0, 0), (0, NHID - NCLASS)))
    b2p = jnp.pad(b2, (0, NHID - NCLASS)).reshape(1, NHID)

    degp = _deg_kernel(dsts)
    h1s, dis = _tc1(degp.reshape(NC, N1, 1), x_pad, W1)
    acc1 = _scatter128(h1s, srcs, dsts)
    h2s = _tc2(acc1, dis, b1.reshape(1, NHID), p.reshape(1, NHID), W2p)
    acc2 = _scatter128(h2s, srcs, dsts)
    return _tc3(acc2, dis, b2p)[:, :NCLASS]


# speedup vs baseline: 28.1191x; 1.2763x over previous
import functools

import jax
import jax.numpy as jnp
from jax import lax
from jax.experimental import pallas as pl
from jax.experimental.pallas import tpu as pltpu
from jax.experimental.pallas import tpu_sc as plsc

N = 10000
NFEAT = 128
NHID = 128
NCLASS = 64
E = 320000

NC = 2
NS = 16
NW = NC * NS
K = 128

N1 = 10112
RPT = N1 // NS
_CHUNKS = [(t * K, K) for t in range(RPT // K)] + [(RPT - RPT % K, RPT % K)]

EP = E + N
NCH = -(-EP // (NW * K))
EPAD = NW * NCH * K

_mesh = plsc.VectorSubcoreMesh(core_axis_name="c", subcore_axis_name="s")


@functools.partial(
    pl.kernel,
    mesh=_mesh,
    out_type=jax.ShapeDtypeStruct((NC * N1,), jnp.float32),
    scratch_types=[
        pltpu.VMEM((NCH, K), jnp.int32),
        pltpu.VMEM((K,), jnp.float32),
        pltpu.VMEM((RPT,), jnp.float32),
        pltpu.VMEM_SHARED((N1,), jnp.float32),
    ],
)
def _deg_kernel(dsts, out, dst_v, ones_v, zld, deg_sh):
    c = lax.axis_index("c")
    s = lax.axis_index("s")
    wid = c * NS + s

    for j in range(K // 16):
        ones_v[pl.ds(j * 16, 16)] = jnp.full((16,), 1.0, jnp.float32)

    def zfill(i, carry):
        zld[pl.ds(i * 16, 16)] = jnp.zeros((16,), jnp.float32)
        return carry

    lax.fori_loop(0, RPT // 16, zfill, 0)
    zld[pl.ds(RPT - 16, 16)] = jnp.zeros((16,), jnp.float32)

    pltpu.sync_copy(zld, deg_sh.at[pl.ds(s * RPT, RPT)])

    pltpu.sync_copy(dsts.at[wid], dst_v)
    plsc.subcore_barrier()

    def body(j, carry):
        pltpu.sync_copy(ones_v, deg_sh.at[dst_v.at[j]], add=True)
        return carry

    lax.fori_loop(0, NCH, body, 0)
    plsc.subcore_barrier()
    pltpu.sync_copy(deg_sh.at[pl.ds(s * RPT, RPT)], zld)
    pltpu.sync_copy(zld, out.at[pl.ds(c * N1 + s * RPT, RPT)])


def _make_scatter_kernel(D):
    @functools.partial(
        pl.kernel,
        mesh=_mesh,
        out_type=jax.ShapeDtypeStruct((NC, N1, D), jnp.float32),
        scratch_types=[
            pltpu.VMEM((2, K), jnp.int32),
            pltpu.VMEM((2, K), jnp.int32),
            pltpu.VMEM((2, K, D), jnp.float32),
            pltpu.VMEM_SHARED((N1, D), jnp.float32),
            pltpu.SemaphoreType.DMA,
            pltpu.SemaphoreType.DMA,
        ],
    )
    def k(table, srcs, dsts, out, src2, dst2, rows2, acc, semg, semi):
        c = lax.axis_index("c")
        s = lax.axis_index("s")
        wid = c * NS + s

        def zfill(i, carry):
            for j in range(D // 16):
                rows2[0, i, pl.ds(j * 16, 16)] = jnp.zeros((16,), jnp.float32)
            return carry

        lax.fori_loop(0, K, zfill, 0)
        for off, n in _CHUNKS:
            pltpu.sync_copy(rows2.at[0, pl.ds(0, n)],
                            acc.at[pl.ds(s * RPT + off, n)])

        pltpu.sync_copy(srcs.at[wid, 0], src2.at[0])
        pltpu.sync_copy(dsts.at[wid, 0], dst2.at[0])
        pltpu.async_copy(table.at[src2.at[0]], rows2.at[0], semg)
        pltpu.async_copy(srcs.at[wid, 1], src2.at[1], semi)
        pltpu.async_copy(dsts.at[wid, 1], dst2.at[1], semi)
        plsc.subcore_barrier()

        def body(j, carry):
            par = lax.rem(j, 2)
            nxt = 1 - par
            pltpu.make_async_copy(table.at[src2.at[par]], rows2.at[par],
                                  semg).wait()

            @pl.when(j + 1 < NCH)
            def _():
                pltpu.make_async_copy(srcs.at[wid, j + 1], src2.at[nxt],
                                      semi).wait()
                pltpu.make_async_copy(dsts.at[wid, j + 1], dst2.at[nxt],
                                      semi).wait()
                pltpu.async_copy(table.at[src2.at[nxt]], rows2.at[nxt], semg)

            pltpu.sync_copy(rows2.at[par], acc.at[dst2.at[par]], add=True)

            @pl.when(j + 2 < NCH)
            def _():
                pltpu.async_copy(srcs.at[wid, j + 2], src2.at[par], semi)
                pltpu.async_copy(dsts.at[wid, j + 2], dst2.at[par], semi)

            return carry

        lax.fori_loop(0, NCH, body, 0)
        plsc.subcore_barrier()
        for off, n in _CHUNKS:
            pltpu.sync_copy(acc.at[pl.ds(s * RPT + off, n)],
                            rows2.at[0, pl.ds(0, n)])
            pltpu.sync_copy(rows2.at[0, pl.ds(0, n)],
                            out.at[c, pl.ds(s * RPT + off, n)])

    return k


_scatter128 = _make_scatter_kernel(NHID)


def _tc1_body(degp_ref, x_ref, w1_ref, h1s_ref, dis_ref):
    deg = degp_ref[0] + degp_ref[1]
    dis = jnp.where(deg > 0.0, lax.rsqrt(deg), 0.0)
    h = jnp.dot(x_ref[...], w1_ref[...], preferred_element_type=jnp.float32)
    h1s_ref[...] = h * dis
    dis_ref[...] = dis


def _tc2_body(accp_ref, dis_ref, b1_ref, p_ref, w2_ref, h2s_ref):
    a = accp_ref[0] + accp_ref[1]
    dis = dis_ref[...]
    z = jnp.maximum(a * dis + b1_ref[...], 0.0) * jnp.clip(p_ref[...], 0.0, 1.0)
    row = lax.broadcasted_iota(jnp.int32, (N1, 1), 0)
    z = jnp.where(row < N, z, 0.0)
    h2s_ref[...] = jnp.dot(z, w2_ref[...],
                           preferred_element_type=jnp.float32) * dis


def _tc3_body(accp_ref, dis_ref, b2_ref, out_ref):
    a = accp_ref[0] + accp_ref[1]
    full = a * dis_ref[...] + b2_ref[...]
    out_ref[...] = full[:N, :]


_tc1 = pl.pallas_call(
    _tc1_body,
    out_shape=(jax.ShapeDtypeStruct((N1, NHID), jnp.float32),
               jax.ShapeDtypeStruct((N1, 1), jnp.float32)),
)

_tc2 = pl.pallas_call(
    _tc2_body,
    out_shape=jax.ShapeDtypeStruct((N1, NHID), jnp.float32),
)

_tc3 = pl.pallas_call(
    _tc3_body,
    out_shape=jax.ShapeDtypeStruct((N, NHID), jnp.float32),
)


def kernel(x, edge_index, W1, b1, W2, b2, p):
    ei = edge_index.astype(jnp.int32)
    src = ei[0]
    dst = ei[1]
    loop = jnp.arange(N, dtype=jnp.int32)
    npad = EPAD - EP
    pad_i = jnp.arange(npad, dtype=jnp.int32)
    pad_src = (pad_i * 997) % N
    pad_dst = N + pad_i % (N1 - N)
    srcs = jnp.concatenate([src, loop, pad_src]).reshape(NW, NCH, K)
    dsts = jnp.concatenate([dst, loop, pad_dst]).reshape(NW, NCH, K)
    x_pad = jnp.pad(x, ((0, N1 - N), (0, 0)))

    W2p = jnp.pad(W2, ((0, 0), (0, NHID - NCLASS)))
    b2p = jnp.pad(b2, (0, NHID - NCLASS)).reshape(1, NHID)

    degp = _deg_kernel(dsts)
    h1s, dis = _tc1(degp.reshape(NC, N1, 1), x_pad, W1)
    acc1 = _scatter128(h1s, srcs, dsts)
    h2s = _tc2(acc1, dis, b1.reshape(1, NHID), p.reshape(1, NHID), W2p)
    acc2 = _scatter128(h2s, srcs, dsts)
    return _tc3(acc2, dis, b2p)[:, :NCLASS]


# speedup vs baseline: 29.6244x; 1.0535x over previous
import functools

import jax
import jax.numpy as jnp
from jax import lax
from jax.experimental import pallas as pl
from jax.experimental.pallas import tpu as pltpu
from jax.experimental.pallas import tpu_sc as plsc

N = 10000
NFEAT = 128
NHID = 128
NCLASS = 64
E = 320000

NC = 2
NS = 16
NW = NC * NS
K = 128

N1 = 10112
RPT = N1 // NS
_CHUNKS = [(t * K, K) for t in range(RPT // K)] + [(RPT - RPT % K, RPT % K)]

NCH = -(-E // (NW * K))
EPAD = NW * NCH * K

_mesh = plsc.VectorSubcoreMesh(core_axis_name="c", subcore_axis_name="s")


@functools.partial(
    pl.kernel,
    mesh=_mesh,
    out_type=jax.ShapeDtypeStruct((NC * N1,), jnp.float32),
    scratch_types=[
        pltpu.VMEM((NCH, K), jnp.int32),
        pltpu.VMEM((K,), jnp.float32),
        pltpu.VMEM((RPT,), jnp.float32),
        pltpu.VMEM_SHARED((N1,), jnp.float32),
        pltpu.SemaphoreType.DMA,
    ],
)
def _deg_kernel(dsts, out, dst_v, ones_v, zld, deg_sh, sem):
    c = lax.axis_index("c")
    s = lax.axis_index("s")
    wid = c * NS + s

    for j in range(K // 16):
        ones_v[pl.ds(j * 16, 16)] = jnp.full((16,), 1.0, jnp.float32)

    def zfill(i, carry):
        zld[pl.ds(i * 16, 16)] = jnp.zeros((16,), jnp.float32)
        return carry

    lax.fori_loop(0, RPT // 16, zfill, 0)
    zld[pl.ds(RPT - 16, 16)] = jnp.zeros((16,), jnp.float32)

    pltpu.sync_copy(zld, deg_sh.at[pl.ds(s * RPT, RPT)])

    pltpu.sync_copy(dsts.at[wid], dst_v)
    plsc.subcore_barrier()

    def body(j, carry):
        pltpu.async_copy(ones_v, deg_sh.at[dst_v.at[j]], sem, add=True)
        return carry

    lax.fori_loop(0, NCH, body, 0)
    pltpu.make_async_copy(dsts.at[wid], dst_v, sem).wait()
    plsc.subcore_barrier()
    pltpu.sync_copy(deg_sh.at[pl.ds(s * RPT, RPT)], zld)
    pltpu.sync_copy(zld, out.at[pl.ds(c * N1 + s * RPT, RPT)])


def _make_scatter_kernel(D):
    @functools.partial(
        pl.kernel,
        mesh=_mesh,
        out_type=jax.ShapeDtypeStruct((NC, N1, D), jnp.float32),
        scratch_types=[
            pltpu.VMEM((2, K), jnp.int32),
            pltpu.VMEM((2, K), jnp.int32),
            pltpu.VMEM((2, K, D), jnp.float32),
            pltpu.VMEM_SHARED((N1, D), jnp.float32),
            pltpu.SemaphoreType.DMA,
            pltpu.SemaphoreType.DMA,
        ],
    )
    def k(table, srcs, dsts, out, src2, dst2, rows2, acc, semg, semi):
        c = lax.axis_index("c")
        s = lax.axis_index("s")
        wid = c * NS + s

        def zfill(i, carry):
            for j in range(D // 16):
                rows2[0, i, pl.ds(j * 16, 16)] = jnp.zeros((16,), jnp.float32)
            return carry

        lax.fori_loop(0, K, zfill, 0)
        for off, n in _CHUNKS:
            pltpu.async_copy(rows2.at[0, pl.ds(0, n)],
                             acc.at[pl.ds(s * RPT + off, n)], semi)
        for off, n in _CHUNKS:
            pltpu.make_async_copy(rows2.at[0, pl.ds(0, n)],
                                  acc.at[pl.ds(s * RPT + off, n)],
                                  semi).wait()

        pltpu.sync_copy(srcs.at[wid, 0], src2.at[0])
        pltpu.sync_copy(dsts.at[wid, 0], dst2.at[0])
        pltpu.async_copy(table.at[src2.at[0]], rows2.at[0], semg)
        pltpu.async_copy(srcs.at[wid, 1], src2.at[1], semi)
        pltpu.async_copy(dsts.at[wid, 1], dst2.at[1], semi)
        plsc.subcore_barrier()

        def body(j, carry):
            par = lax.rem(j, 2)
            nxt = 1 - par
            pltpu.make_async_copy(table.at[src2.at[par]], rows2.at[par],
                                  semg).wait()

            @pl.when(j + 1 < NCH)
            def _():
                pltpu.make_async_copy(srcs.at[wid, j + 1], src2.at[nxt],
                                      semi).wait()
                pltpu.make_async_copy(dsts.at[wid, j + 1], dst2.at[nxt],
                                      semi).wait()
                pltpu.async_copy(table.at[src2.at[nxt]], rows2.at[nxt], semg)

            pltpu.sync_copy(rows2.at[par], acc.at[dst2.at[par]], add=True)

            @pl.when(j + 2 < NCH)
            def _():
                pltpu.async_copy(srcs.at[wid, j + 2], src2.at[par], semi)
                pltpu.async_copy(dsts.at[wid, j + 2], dst2.at[par], semi)

            return carry

        lax.fori_loop(0, NCH, body, 0)
        plsc.subcore_barrier()
        def _wr(i):
            off, n = _CHUNKS[i]
            return (rows2.at[i % 2, pl.ds(0, n)],
                    out.at[c, pl.ds(s * RPT + off, n)])

        for i, (off, n) in enumerate(_CHUNKS):
            if i >= 2:
                pltpu.make_async_copy(*_wr(i - 2), semi).wait()
            pltpu.sync_copy(acc.at[pl.ds(s * RPT + off, n)],
                            rows2.at[i % 2, pl.ds(0, n)])
            pltpu.async_copy(*_wr(i), semi)
        for i in range(max(0, len(_CHUNKS) - 2), len(_CHUNKS)):
            pltpu.make_async_copy(*_wr(i), semi).wait()

    return k


_scatter128 = _make_scatter_kernel(NHID)


def _tc1_body(degp_ref, x_ref, w1_ref, h1s_ref, dis_ref):
    deg = degp_ref[0] + degp_ref[1] + 1.0
    dis = jnp.where(deg > 0.0, lax.rsqrt(deg), 0.0)
    h = jnp.dot(x_ref[...], w1_ref[...], preferred_element_type=jnp.float32)
    h1s_ref[...] = h * dis
    dis_ref[...] = dis


def _tc2_body(accp_ref, h1s_ref, dis_ref, b1_ref, p_ref, w2_ref, h2s_ref):
    a = accp_ref[0] + accp_ref[1] + h1s_ref[...]
    dis = dis_ref[...]
    z = jnp.maximum(a * dis + b1_ref[...], 0.0) * jnp.clip(p_ref[...], 0.0, 1.0)
    row = lax.broadcasted_iota(jnp.int32, (N1, 1), 0)
    z = jnp.where(row < N, z, 0.0)
    h2s_ref[...] = jnp.dot(z, w2_ref[...],
                           preferred_element_type=jnp.float32) * dis


def _tc3_body(accp_ref, h2s_ref, dis_ref, b2_ref, out_ref):
    a = accp_ref[0] + accp_ref[1] + h2s_ref[...]
    full = a * dis_ref[...] + b2_ref[...]
    out_ref[...] = full[:N, :]


_tc1 = pl.pallas_call(
    _tc1_body,
    out_shape=(jax.ShapeDtypeStruct((N1, NHID), jnp.float32),
               jax.ShapeDtypeStruct((N1, 1), jnp.float32)),
)

_tc2 = pl.pallas_call(
    _tc2_body,
    out_shape=jax.ShapeDtypeStruct((N1, NHID), jnp.float32),
)

_tc3 = pl.pallas_call(
    _tc3_body,
    out_shape=jax.ShapeDtypeStruct((N, NHID), jnp.float32),
)


def kernel(x, edge_index, W1, b1, W2, b2, p):
    ei = edge_index.astype(jnp.int32)
    src = ei[0]
    dst = ei[1]
    npad = EPAD - E
    pad_i = jnp.arange(npad, dtype=jnp.int32)
    pad_src = (pad_i * 997) % N
    pad_dst = N + pad_i % (N1 - N)
    srcs = jnp.concatenate([src, pad_src]).reshape(NW, NCH, K)
    dsts = jnp.concatenate([dst, pad_dst]).reshape(NW, NCH, K)
    x_pad = jnp.pad(x, ((0, N1 - N), (0, 0)))

    W2p = jnp.pad(W2, ((0, 0), (0, NHID - NCLASS)))
    b2p = jnp.pad(b2, (0, NHID - NCLASS)).reshape(1, NHID)

    degp = _deg_kernel(dsts)
    h1s, dis = _tc1(degp.reshape(NC, N1, 1), x_pad, W1)
    acc1 = _scatter128(h1s, srcs, dsts)
    h2s = _tc2(acc1, h1s, dis, b1.reshape(1, NHID), p.reshape(1, NHID), W2p)
    acc2 = _scatter128(h2s, srcs, dsts)
    return _tc3(acc2, h2s, dis, b2p)[:, :NCLASS]


# speedup vs baseline: 31.0265x; 1.0473x over previous
import functools

import jax
import jax.numpy as jnp
from jax import lax
from jax.experimental import pallas as pl
from jax.experimental.pallas import tpu as pltpu
from jax.experimental.pallas import tpu_sc as plsc

N = 10000
NFEAT = 128
NHID = 128
NCLASS = 64
E = 320000

NC = 2
NS = 16
NW = NC * NS
K = 128

N1 = 10112
RPT = N1 // NS
_CHUNKS = [(t * K, K) for t in range(RPT // K)] + [(RPT - RPT % K, RPT % K)]

NCH = -(-E // (NW * K))
EPAD = NW * NCH * K

_mesh = plsc.VectorSubcoreMesh(core_axis_name="c", subcore_axis_name="s")


@functools.partial(
    pl.kernel,
    mesh=_mesh,
    out_type=jax.ShapeDtypeStruct((NC * N1,), jnp.float32),
    scratch_types=[
        pltpu.VMEM((NCH, K), jnp.int32),
        pltpu.VMEM((K,), jnp.float32),
        pltpu.VMEM((RPT,), jnp.float32),
        pltpu.VMEM_SHARED((N1,), jnp.float32),
        pltpu.SemaphoreType.DMA,
    ],
)
def _deg_kernel(dsts, out, dst_v, ones_v, zld, deg_sh, sem):
    c = lax.axis_index("c")
    s = lax.axis_index("s")
    wid = c * NS + s

    for j in range(K // 16):
        ones_v[pl.ds(j * 16, 16)] = jnp.full((16,), 1.0, jnp.float32)

    def zfill(i, carry):
        zld[pl.ds(i * 16, 16)] = jnp.zeros((16,), jnp.float32)
        return carry

    lax.fori_loop(0, RPT // 16, zfill, 0)
    zld[pl.ds(RPT - 16, 16)] = jnp.zeros((16,), jnp.float32)

    pltpu.sync_copy(zld, deg_sh.at[pl.ds(s * RPT, RPT)])

    pltpu.sync_copy(dsts.at[wid], dst_v)
    plsc.subcore_barrier()

    def body(j, carry):
        pltpu.async_copy(ones_v, deg_sh.at[dst_v.at[j]], sem, add=True)
        return carry

    lax.fori_loop(0, NCH, body, 0)
    pltpu.make_async_copy(dsts.at[wid], dst_v, sem).wait()
    plsc.subcore_barrier()
    pltpu.sync_copy(deg_sh.at[pl.ds(s * RPT, RPT)], zld)
    pltpu.sync_copy(zld, out.at[pl.ds(c * N1 + s * RPT, RPT)])


def _make_scatter_kernel(D):
    @functools.partial(
        pl.kernel,
        mesh=_mesh,
        out_type=jax.ShapeDtypeStruct((NC, N1, D), jnp.float32),
        scratch_types=[
            pltpu.VMEM((2, K), jnp.int32),
            pltpu.VMEM((2, K), jnp.int32),
            pltpu.VMEM((2, K, D), jnp.float32),
            pltpu.VMEM_SHARED((N1, D), jnp.float32),
            pltpu.SemaphoreType.DMA,
            pltpu.SemaphoreType.DMA,
        ],
    )
    def k(table, srcs, dsts, out, src2, dst2, rows2, acc, semg, semi):
        c = lax.axis_index("c")
        s = lax.axis_index("s")
        wid = c * NS + s

        def zfill(i, carry):
            for j in range(D // 16):
                rows2[0, i, pl.ds(j * 16, 16)] = jnp.zeros((16,), jnp.float32)
            return carry

        lax.fori_loop(0, K, zfill, 0)
        for off, n in _CHUNKS:
            pltpu.async_copy(rows2.at[0, pl.ds(0, n)],
                             acc.at[pl.ds(s * RPT + off, n)], semi)
        for off, n in _CHUNKS:
            pltpu.make_async_copy(rows2.at[0, pl.ds(0, n)],
                                  acc.at[pl.ds(s * RPT + off, n)],
                                  semi).wait()

        pltpu.sync_copy(srcs.at[wid, 0], src2.at[0])
        pltpu.sync_copy(dsts.at[wid, 0], dst2.at[0])
        pltpu.async_copy(table.at[src2.at[0]], rows2.at[0], semg)
        pltpu.async_copy(srcs.at[wid, 1], src2.at[1], semi)
        pltpu.async_copy(dsts.at[wid, 1], dst2.at[1], semi)
        plsc.subcore_barrier()

        def body(j, carry):
            par = lax.rem(j, 2)
            nxt = 1 - par
            pltpu.make_async_copy(table.at[src2.at[par]], rows2.at[par],
                                  semg).wait()

            @pl.when(j + 1 < NCH)
            def _():
                pltpu.make_async_copy(srcs.at[wid, j + 1], src2.at[nxt],
                                      semi).wait()
                pltpu.make_async_copy(dsts.at[wid, j + 1], dst2.at[nxt],
                                      semi).wait()
                pltpu.async_copy(table.at[src2.at[nxt]], rows2.at[nxt], semg)

            pltpu.sync_copy(rows2.at[par], acc.at[dst2.at[par]], add=True)

            @pl.when(j + 2 < NCH)
            def _():
                pltpu.async_copy(srcs.at[wid, j + 2], src2.at[par], semi)
                pltpu.async_copy(dsts.at[wid, j + 2], dst2.at[par], semi)

            return carry

        lax.fori_loop(0, NCH, body, 0)
        plsc.subcore_barrier()
        def _wr(i):
            off, n = _CHUNKS[i]
            return (rows2.at[i % 2, pl.ds(0, n)],
                    out.at[c, pl.ds(s * RPT + off, n)])

        for i, (off, n) in enumerate(_CHUNKS):
            if i >= 2:
                pltpu.make_async_copy(*_wr(i - 2), semi).wait()
            pltpu.sync_copy(acc.at[pl.ds(s * RPT + off, n)],
                            rows2.at[i % 2, pl.ds(0, n)])
            pltpu.async_copy(*_wr(i), semi)
        for i in range(max(0, len(_CHUNKS) - 2), len(_CHUNKS)):
            pltpu.make_async_copy(*_wr(i), semi).wait()

    return k


_scatter128 = _make_scatter_kernel(NHID)


def _tc1_body(degp_ref, x_ref, w1_ref, h1s_ref, dis_ref):
    deg = (degp_ref[pl.ds(0, N1)] + degp_ref[pl.ds(N1, N1)] + 1.0)
    dis = jnp.where(deg > 0.0, lax.rsqrt(deg), 0.0).reshape(N1, 1)
    h = jnp.dot(x_ref[...], w1_ref[...], preferred_element_type=jnp.float32)
    h1s_ref[pl.ds(0, N), :] = h * dis[:N]
    h1s_ref[pl.ds(N, N1 - N), :] = jnp.zeros((N1 - N, NHID), jnp.float32)
    dis_ref[...] = dis


def _tc2_body(accp_ref, h1s_ref, dis_ref, b1_ref, p_ref, w2_ref, h2s_ref):
    a = accp_ref[0] + accp_ref[1] + h1s_ref[...]
    dis = dis_ref[...]
    z = jnp.maximum(a * dis + b1_ref[...], 0.0) * jnp.clip(p_ref[...], 0.0, 1.0)
    row = lax.broadcasted_iota(jnp.int32, (N1, 1), 0)
    z = jnp.where(row < N, z, 0.0)
    h2s_ref[...] = jnp.dot(z, w2_ref[...],
                           preferred_element_type=jnp.float32) * dis


def _tc3_body(accp_ref, h2s_ref, dis_ref, b2_ref, out_ref):
    a = accp_ref[0] + accp_ref[1] + h2s_ref[...]
    full = a * dis_ref[...] + b2_ref[...]
    out_ref[...] = full[:N, :NCLASS]


_tc1 = pl.pallas_call(
    _tc1_body,
    out_shape=(jax.ShapeDtypeStruct((N1, NHID), jnp.float32),
               jax.ShapeDtypeStruct((N1, 1), jnp.float32)),
)

_tc2 = pl.pallas_call(
    _tc2_body,
    out_shape=jax.ShapeDtypeStruct((N1, NHID), jnp.float32),
)

_tc3 = pl.pallas_call(
    _tc3_body,
    out_shape=jax.ShapeDtypeStruct((N, NCLASS), jnp.float32),
)


def kernel(x, edge_index, W1, b1, W2, b2, p):
    ei = edge_index.astype(jnp.int32)
    src = ei[0]
    dst = ei[1]
    npad = EPAD - E
    pad_i = jnp.arange(npad, dtype=jnp.int32)
    pad_src = (pad_i * 997) % N
    pad_dst = N + pad_i % (N1 - N)
    srcs = jnp.concatenate([src, pad_src]).reshape(NW, NCH, K)
    dsts = jnp.concatenate([dst, pad_dst]).reshape(NW, NCH, K)

    W2p = jnp.pad(W2, ((0, 0), (0, NHID - NCLASS)))
    b2p = jnp.pad(b2, (0, NHID - NCLASS)).reshape(1, NHID)

    degp = _deg_kernel(dsts)
    h1s, dis = _tc1(degp, x, W1)
    acc1 = _scatter128(h1s, srcs, dsts)
    h2s = _tc2(acc1, h1s, dis, b1.reshape(1, NHID), p.reshape(1, NHID), W2p)
    acc2 = _scatter128(h2s, srcs, dsts)
    return _tc3(acc2, h2s, dis, b2p)


# speedup vs baseline: 39.2026x; 1.2635x over previous
import functools

import jax
import jax.numpy as jnp
from jax import lax
from jax.experimental import pallas as pl
from jax.experimental.pallas import tpu as pltpu
from jax.experimental.pallas import tpu_sc as plsc

N = 10000
NFEAT = 128
NHID = 128
NCLASS = 64
E = 320000

NC = 2
NS = 16
NW = NC * NS
K = 128

N1 = 10112
RPT = N1 // NS
_CHUNKS = [(t * K, K) for t in range(RPT // K)] + [(RPT - RPT % K, RPT % K)]

NCH = -(-E // (NW * K))
EPAD = NW * NCH * K

_mesh = plsc.VectorSubcoreMesh(core_axis_name="c", subcore_axis_name="s")


@functools.partial(
    pl.kernel,
    mesh=_mesh,
    out_type=jax.ShapeDtypeStruct((NC * N1,), jnp.float32),
    scratch_types=[
        pltpu.VMEM((NCH, K), jnp.int32),
        pltpu.VMEM((K,), jnp.float32),
        pltpu.VMEM((RPT,), jnp.float32),
        pltpu.VMEM_SHARED((N1,), jnp.float32),
        pltpu.SemaphoreType.DMA,
    ],
)
def _deg_kernel(dsts, out, dst_v, ones_v, zld, deg_sh, sem):
    c = lax.axis_index("c")
    s = lax.axis_index("s")
    wid = c * NS + s

    for j in range(K // 16):
        ones_v[pl.ds(j * 16, 16)] = jnp.full((16,), 1.0, jnp.float32)

    def zfill(i, carry):
        zld[pl.ds(i * 16, 16)] = jnp.zeros((16,), jnp.float32)
        return carry

    lax.fori_loop(0, RPT // 16, zfill, 0)
    zld[pl.ds(RPT - 16, 16)] = jnp.zeros((16,), jnp.float32)

    pltpu.sync_copy(zld, deg_sh.at[pl.ds(s * RPT, RPT)])

    pltpu.sync_copy(dsts.at[wid], dst_v)
    plsc.subcore_barrier()

    def body(j, carry):
        pltpu.async_copy(ones_v, deg_sh.at[dst_v.at[j]], sem, add=True)
        return carry

    lax.fori_loop(0, NCH, body, 0)
    pltpu.make_async_copy(dsts.at[wid], dst_v, sem).wait()
    plsc.subcore_barrier()
    pltpu.sync_copy(deg_sh.at[pl.ds(s * RPT, RPT)], zld)
    pltpu.sync_copy(zld, out.at[pl.ds(c * N1 + s * RPT, RPT)])


def _make_scatter_kernel(D):
    @functools.partial(
        pl.kernel,
        mesh=_mesh,
        out_type=jax.ShapeDtypeStruct((NC, N1, D), jnp.float32),
        scratch_types=[
            pltpu.VMEM((3, K), jnp.int32),
            pltpu.VMEM((4, K), jnp.int32),
            pltpu.VMEM((3, K, D), jnp.float32),
            pltpu.VMEM_SHARED((N1, D), jnp.float32),
            pltpu.SemaphoreType.DMA,
            pltpu.SemaphoreType.DMA,
            pltpu.SemaphoreType.DMA,
        ],
    )
    def k(table, srcs, dsts, out, src3, dst4, rows3, acc, semg, semi, sems):
        c = lax.axis_index("c")
        s = lax.axis_index("s")
        wid = c * NS + s

        def zfill(i, carry):
            for j in range(D // 16):
                rows3[0, i, pl.ds(j * 16, 16)] = jnp.zeros((16,), jnp.float32)
            return carry

        lax.fori_loop(0, K, zfill, 0)
        for off, n in _CHUNKS:
            pltpu.async_copy(rows3.at[0, pl.ds(0, n)],
                             acc.at[pl.ds(s * RPT + off, n)], semi)
        for off, n in _CHUNKS:
            pltpu.make_async_copy(rows3.at[0, pl.ds(0, n)],
                                  acc.at[pl.ds(s * RPT + off, n)],
                                  semi).wait()

        pltpu.sync_copy(srcs.at[wid, 0], src3.at[0])
        pltpu.sync_copy(dsts.at[wid, 0], dst4.at[0])
        pltpu.async_copy(table.at[src3.at[0]], rows3.at[0], semg)
        pltpu.sync_copy(srcs.at[wid, 1], src3.at[1])
        pltpu.sync_copy(dsts.at[wid, 1], dst4.at[1])
        pltpu.async_copy(table.at[src3.at[1]], rows3.at[1], semg)
        pltpu.async_copy(srcs.at[wid, 2], src3.at[2], semi)
        pltpu.async_copy(dsts.at[wid, 2], dst4.at[2], semi)
        plsc.subcore_barrier()

        def body(j, carry):
            b = lax.rem(j, 3)
            b2 = lax.rem(j + 2, 3)
            pltpu.make_async_copy(table.at[src3.at[b]], rows3.at[b],
                                  semg).wait()
            pltpu.async_copy(rows3.at[b], acc.at[dst4.at[lax.rem(j, 4)]],
                             sems, add=True)

            @pl.when(j + 2 < NCH)
            def _():
                pltpu.make_async_copy(srcs.at[wid, j + 2], src3.at[b2],
                                      semi).wait()
                pltpu.make_async_copy(dsts.at[wid, j + 2],
                                      dst4.at[lax.rem(j + 2, 4)],
                                      semi).wait()

                @pl.when(j >= 1)
                def _():
                    pltpu.make_async_copy(rows3.at[b2],
                                          acc.at[dst4.at[lax.rem(j + 3, 4)]],
                                          sems).wait()

                pltpu.async_copy(table.at[src3.at[b2]], rows3.at[b2], semg)

                @pl.when(j + 3 < NCH)
                def _():
                    pltpu.async_copy(srcs.at[wid, j + 3], src3.at[b], semi)
                    pltpu.async_copy(dsts.at[wid, j + 3],
                                     dst4.at[lax.rem(j + 3, 4)], semi)

            return carry

        lax.fori_loop(0, NCH, body, 0)
        for _ in range(3):
            pltpu.make_async_copy(rows3.at[0], acc.at[dst4.at[0]],
                                  sems).wait()
        plsc.subcore_barrier()
        def _wr(i):
            off, n = _CHUNKS[i]
            return (rows3.at[i % 2, pl.ds(0, n)],
                    out.at[c, pl.ds(s * RPT + off, n)])

        for i, (off, n) in enumerate(_CHUNKS):
            if i >= 2:
                pltpu.make_async_copy(*_wr(i - 2), semi).wait()
            pltpu.sync_copy(acc.at[pl.ds(s * RPT + off, n)],
                            rows3.at[i % 2, pl.ds(0, n)])
            pltpu.async_copy(*_wr(i), semi)
        for i in range(max(0, len(_CHUNKS) - 2), len(_CHUNKS)):
            pltpu.make_async_copy(*_wr(i), semi).wait()

    return k


_scatter128 = _make_scatter_kernel(NHID)


def _tc1_body(degp_ref, x_ref, w1_ref, h1s_ref, dis_ref):
    deg = (degp_ref[pl.ds(0, N1)] + degp_ref[pl.ds(N1, N1)] + 1.0)
    dis = jnp.where(deg > 0.0, lax.rsqrt(deg), 0.0).reshape(N1, 1)
    h = jnp.dot(x_ref[...], w1_ref[...], preferred_element_type=jnp.float32)
    h1s_ref[pl.ds(0, N), :] = h * dis[:N]
    h1s_ref[pl.ds(N, N1 - N), :] = jnp.zeros((N1 - N, NHID), jnp.float32)
    dis_ref[...] = dis


def _tc2_body(accp_ref, h1s_ref, dis_ref, b1_ref, p_ref, w2_ref, h2s_ref):
    a = accp_ref[0] + accp_ref[1] + h1s_ref[...]
    dis = dis_ref[...]
    z = jnp.maximum(a * dis + b1_ref[...], 0.0) * jnp.clip(p_ref[...], 0.0, 1.0)
    row = lax.broadcasted_iota(jnp.int32, (N1, 1), 0)
    z = jnp.where(row < N, z, 0.0)
    h2s_ref[...] = jnp.dot(z, w2_ref[...],
                           preferred_element_type=jnp.float32) * dis


def _tc3_body(accp_ref, h2s_ref, dis_ref, b2_ref, out_ref):
    a = accp_ref[0] + accp_ref[1] + h2s_ref[...]
    full = a * dis_ref[...] + b2_ref[...]
    out_ref[...] = full[:N, :NCLASS]


_tc1 = pl.pallas_call(
    _tc1_body,
    out_shape=(jax.ShapeDtypeStruct((N1, NHID), jnp.float32),
               jax.ShapeDtypeStruct((N1, 1), jnp.float32)),
)

_tc2 = pl.pallas_call(
    _tc2_body,
    out_shape=jax.ShapeDtypeStruct((N1, NHID), jnp.float32),
)

_tc3 = pl.pallas_call(
    _tc3_body,
    out_shape=jax.ShapeDtypeStruct((N, NCLASS), jnp.float32),
)


def kernel(x, edge_index, W1, b1, W2, b2, p):
    ei = edge_index.astype(jnp.int32)
    src = ei[0]
    dst = ei[1]
    npad = EPAD - E
    pad_i = jnp.arange(npad, dtype=jnp.int32)
    pad_src = (pad_i * 997) % N
    pad_dst = N + pad_i % (N1 - N)
    srcs = jnp.concatenate([src, pad_src]).reshape(NW, NCH, K)
    dsts = jnp.concatenate([dst, pad_dst]).reshape(NW, NCH, K)

    W2p = jnp.pad(W2, ((0, 0), (0, NHID - NCLASS)))
    b2p = jnp.pad(b2, (0, NHID - NCLASS)).reshape(1, NHID)

    degp = _deg_kernel(dsts)
    h1s, dis = _tc1(degp, x, W1)
    acc1 = _scatter128(h1s, srcs, dsts)
    h2s = _tc2(acc1, h1s, dis, b1.reshape(1, NHID), p.reshape(1, NHID), W2p)
    acc2 = _scatter128(h2s, srcs, dsts)
    return _tc3(acc2, h2s, dis, b2p)
